# baseline (speedup 1.0000x reference)
import jax
import jax.numpy as jnp
from jax.experimental import pallas as pl

_CHUNK = 8


def _cindex_kernel(tr_ref, er_ref, sr_ref, tc_ref, ec_ref, sc_ref, out_ref):
    t_b = jnp.exp(tr_ref[...])
    e_b = er_ref[...] > 0.0
    s_b = sr_ref[...]
    n = t_b.shape[1]

    def body(c, carry):
        con, tie, tot = carry
        base = c * _CHUNK
        t_a = jnp.exp(tc_ref[pl.ds(base, _CHUNK), :])
        e_a = ec_ref[pl.ds(base, _CHUNK), :] > 0.0
        s_a = sc_ref[pl.ds(base, _CHUNK), :]

        lt = t_a < t_b
        eq = t_a == t_b
        cmp = e_a & (lt | (eq & (~e_b)))
        conm = cmp & (s_b < s_a)
        tiem = cmp & (jnp.abs(s_b - s_a) <= 1e-8)

        con = con + jnp.sum(conm.astype(jnp.int32))
        tie = tie + jnp.sum(tiem.astype(jnp.int32))
        tot = tot + jnp.sum(cmp.astype(jnp.int32))
        return con, tie, tot

    zero = jnp.int32(0)
    con, tie, tot = jax.lax.fori_loop(0, n // _CHUNK, body, (zero, zero, zero))

    tie_f = tie.astype(jnp.float32)
    tot_f = tot.astype(jnp.float32)
    disc_f = (tot - con - tie).astype(jnp.float32)
    loss = 1.0 - (disc_f + 0.5 * tie_f) / (tot_f + 1e-7)
    out_ref[...] = jnp.broadcast_to(loss, (1, 1))


def kernel(event_indicator, event_time, estimate):
    x = jnp.asarray(event_time, jnp.float32).reshape(-1)
    s = jnp.asarray(estimate, jnp.float32).reshape(-1)
    e = jnp.asarray(event_indicator).astype(jnp.float32).reshape(-1)
    n = x.shape[0]

    out = pl.pallas_call(
        _cindex_kernel,
        out_shape=jax.ShapeDtypeStruct((1, 1), jnp.float32),
    )(
        x.reshape(1, n), e.reshape(1, n), s.reshape(1, n),
        x.reshape(n, 1), e.reshape(n, 1), s.reshape(n, 1),
    )
    return out[0, 0]


# speedup vs baseline: 1.9611x; 1.9611x over previous
import jax
import jax.numpy as jnp
from jax.experimental import pallas as pl

_CHUNK = 8
_SIGN = -2147483648


def _cindex_kernel(tr_ref, er_ref, sr_ref, tc_ref, ec_ref, sc_ref, out_ref):
    n = tr_ref.shape[1]
    t_b = jnp.exp(tr_ref[...])
    e_b = er_ref[...].astype(jnp.int32)
    s_b = sr_ref[...]
    bits_b = jax.lax.bitcast_convert_type(t_b, jnp.int32)
    key_b = ((bits_b << 1) | (1 - e_b)) ^ _SIGN

    def body(c, acc):
        base = c * _CHUNK
        t_a = jnp.exp(tc_ref[pl.ds(base, _CHUNK), :])
        e_a = ec_ref[pl.ds(base, _CHUNK), :] > 0.0
        s_a = sc_ref[pl.ds(base, _CHUNK), :]
        bits_a = jax.lax.bitcast_convert_type(t_a, jnp.int32)
        key_a = (bits_a << 1) ^ _SIGN

        cmp = e_a & (key_a < key_b)
        conm = s_b < s_a
        d = s_b - s_a
        tiem = jnp.abs(d) <= 1e-8
        w = (1 + jnp.where(conm, 1024, 0)) + jnp.where(tiem, 1048576, 0)
        return acc + jnp.where(cmp, w, 0)

    acc0 = jnp.zeros((_CHUNK, n), jnp.int32)
    acc = jax.lax.fori_loop(0, n // _CHUNK, body, acc0)

    tot = jnp.sum(acc & 1023)
    con = jnp.sum((acc >> 10) & 1023)
    tie = jnp.sum(acc >> 20)

    tie_f = tie.astype(jnp.float32)
    tot_f = tot.astype(jnp.float32)
    disc_f = (tot - con - tie).astype(jnp.float32)
    loss = 1.0 - (disc_f + 0.5 * tie_f) / (tot_f + 1e-7)
    out_ref[...] = jnp.broadcast_to(loss, (1, 1))


def kernel(event_indicator, event_time, estimate):
    x = jnp.asarray(event_time, jnp.float32).reshape(-1)
    s = jnp.asarray(estimate, jnp.float32).reshape(-1)
    e = jnp.asarray(event_indicator).astype(jnp.float32).reshape(-1)
    n = x.shape[0]

    out = pl.pallas_call(
        _cindex_kernel,
        out_shape=jax.ShapeDtypeStruct((1, 1), jnp.float32),
    )(
        x.reshape(1, n), e.reshape(1, n), s.reshape(1, n),
        x.reshape(n, 1), e.reshape(n, 1), s.reshape(n, 1),
    )
    return out[0, 0]


# speedup vs baseline: 5.0767x; 2.5887x over previous
import jax
import jax.numpy as jnp
from jax.experimental import pallas as pl
from jax.experimental.pallas import tpu as pltpu

_SIGN = -2147483648
_IMAX = 2147483647


def _key_kernel(x_ref, e_ref, key_ref):
    t = jnp.exp(x_ref[...])
    bits = jax.lax.bitcast_convert_type(t, jnp.int32)
    key = (bits << 1) ^ _SIGN
    key_ref[...] = jnp.where(e_ref[...] > 0.0, key, _IMAX)


def _count_kernel(xb_ref, eb_ref, sb_ref, ka_ref, sa_ref, out_ref):
    rows, cols = xb_ref.shape
    n = rows * cols
    t_b = jnp.exp(xb_ref[...])
    bits_b = jax.lax.bitcast_convert_type(t_b, jnp.int32)
    e_b = eb_ref[...].astype(jnp.int32)
    key_b = ((bits_b << 1) | (1 - e_b)) ^ _SIGN
    s_b = sb_ref[...]

    def body(a, carry):
        acc1, acc2 = carry
        ka = ka_ref[a]
        sa = sa_ref[a]
        cmp = ka < key_b
        conm = s_b < sa
        tiem = jnp.abs(s_b - sa) <= 1e-8
        acc1 = acc1 + jnp.where(cmp, jnp.where(conm, 65537, 1), 0)
        acc2 = acc2 + jnp.where(cmp & tiem, 1, 0)
        return acc1, acc2

    zeros = jnp.zeros((rows, cols), jnp.int32)
    acc1, acc2 = jax.lax.fori_loop(0, n, body, (zeros, zeros), unroll=8)

    tot = jnp.sum(acc1 & 65535)
    con = jnp.sum(acc1 >> 16)
    tie = jnp.sum(acc2)

    tie_f = tie.astype(jnp.float32)
    tot_f = tot.astype(jnp.float32)
    disc_f = (tot - con - tie).astype(jnp.float32)
    loss = 1.0 - (disc_f + 0.5 * tie_f) / (tot_f + 1e-7)
    out_ref[...] = jnp.broadcast_to(loss, (1, 1))


def kernel(event_indicator, event_time, estimate):
    x = jnp.asarray(event_time, jnp.float32).reshape(-1)
    s = jnp.asarray(estimate, jnp.float32).reshape(-1)
    e = jnp.asarray(event_indicator).astype(jnp.float32).reshape(-1)
    n = x.shape[0]
    rows, cols = 8, n // 8

    key_a = pl.pallas_call(
        _key_kernel,
        out_shape=jax.ShapeDtypeStruct((1, n), jnp.int32),
    )(x.reshape(1, n), e.reshape(1, n))

    out = pl.pallas_call(
        _count_kernel,
        in_specs=[
            pl.BlockSpec(memory_space=pltpu.VMEM),
            pl.BlockSpec(memory_space=pltpu.VMEM),
            pl.BlockSpec(memory_space=pltpu.VMEM),
            pl.BlockSpec(memory_space=pltpu.SMEM),
            pl.BlockSpec(memory_space=pltpu.SMEM),
        ],
        out_shape=jax.ShapeDtypeStruct((1, 1), jnp.float32),
    )(
        x.reshape(rows, cols), e.reshape(rows, cols), s.reshape(rows, cols),
        key_a.reshape(n), s.reshape(n),
    )
    return out[0, 0]
